# baseline (speedup 1.0000x reference)
import jax
import jax.numpy as jnp
from jax.experimental import pallas as pl
from jax.experimental.pallas import tpu as pltpu
from jax.experimental.pallas import tpu_sc as plsc

N = 4096
D = 128
K = 2048

IB = 512
AB = 512
MB = 256
WIN = 128


def _rank_kernel(v_row_ref, v_col_ref, h_ref, rank_i_ref, hv_ref):
    pid = pl.program_id(0)
    v_row = v_row_ref[...]
    v_col = v_col_ref[...]
    gt = v_row > v_col
    eq = v_row == v_col
    jj = jax.lax.broadcasted_iota(jnp.int32, (IB, N), 1)
    ii = jax.lax.broadcasted_iota(jnp.int32, (IB, N), 0) + pid * IB
    cnt = gt.astype(jnp.float32) + (eq & (jj < ii)).astype(jnp.float32)
    rank = jnp.sum(cnt, axis=1, keepdims=True)
    rank_i_ref[...] = rank.astype(jnp.int32)
    hv_ref[...] = h_ref[...] * v_col


def _build_a_kernel(rank_row_ref, g_ref, a_ref, m_scratch):
    pid = pl.program_id(0)

    @pl.when(pid == 0)
    def _():
        rank_row = rank_row_ref[...]
        for rc in range(K // IB):
            rr = jax.lax.broadcasted_iota(jnp.int32, (IB, N), 0) + (rc * IB)
            m_scratch[rc * IB:(rc + 1) * IB, :] = (rank_row == rr).astype(
                jnp.bfloat16)

    gb = (g_ref[...] != 0).astype(jnp.bfloat16)
    a_ref[...] = jnp.dot(m_scratch[...], gb,
                         preferred_element_type=jnp.float32).astype(jnp.bfloat16)


def _twohop_kernel(a_ref, g_ref, rank_col_ref, out_ref, idx_ref, idxacc):
    m = pl.program_id(0)
    nm = pl.num_programs(0)

    @pl.when(m == 0)
    def _():
        out_ref[...] = jnp.zeros_like(out_ref)
        idxacc[...] = jnp.zeros_like(idxacc)

    gb = (g_ref[...] != 0).astype(jnp.bfloat16)
    d = jnp.dot(a_ref[...], gb, preferred_element_type=jnp.float32)
    dbool = (d != 0.0).astype(jnp.bfloat16)

    rank_col = rank_col_ref[...]
    jcols = jax.lax.broadcasted_iota(jnp.int32, (MB, K), 1)
    p_f = (rank_col == jcols).astype(jnp.float32)
    out_ref[...] += jnp.dot(dbool, p_f.astype(jnp.bfloat16),
                            preferred_element_type=jnp.float32)

    miota = (jax.lax.broadcasted_iota(jnp.int32, (MB, 1), 0)
             + m * MB).astype(jnp.float32)
    idxacc[...] += jnp.sum(p_f * miota, axis=0, keepdims=True)
    idx_ref[...] = idxacc[...].astype(jnp.int32)

    @pl.when(m == nm - 1)
    def _():
        c = out_ref[...]
        ones = jnp.ones((1, K), jnp.float32)
        deg = jax.lax.dot_general(ones, c, (((1,), (1,)), ((), ())),
                                  preferred_element_type=jnp.float32)
        out_ref[...] = c / deg


def _sc_scatter_rows(hv, rank_i32):
    rank2 = rank_i32.reshape(1, N)
    mesh = plsc.VectorSubcoreMesh(core_axis_name="c", subcore_axis_name="s")

    @pl.kernel(out_type=jax.ShapeDtypeStruct((N, D), jnp.float32), mesh=mesh)
    def k(hv_hbm, r_hbm, o_hbm):
        def body(hv_vmem, r_vmem):
            pltpu.sync_copy(hv_vmem, o_hbm.at[r_vmem.at[0]])

        pltpu.emit_pipeline(
            body,
            grid=(N // WIN,),
            in_specs=[pl.BlockSpec((WIN, D), lambda i: (i, 0)),
                      pl.BlockSpec((1, WIN), lambda i: (0, i))],
            out_specs=[],
            core_axis_name=("c", "s"),
            dimension_semantics=(pltpu.PARALLEL,),
        )(hv_hbm, r_hbm)

    return k(hv, rank2)


def kernel(g, h, W, b):
    weights = (h @ W + b).squeeze(-1)
    v = jax.nn.sigmoid(weights)
    v_row = v.reshape(1, N)
    v_col = v.reshape(N, 1)

    rank_i, hv = pl.pallas_call(
        _rank_kernel,
        grid=(N // IB,),
        in_specs=[
            pl.BlockSpec((1, N), lambda i: (0, 0)),
            pl.BlockSpec((IB, 1), lambda i: (i, 0)),
            pl.BlockSpec((IB, D), lambda i: (i, 0)),
        ],
        out_specs=[
            pl.BlockSpec((IB, 1), lambda i: (i, 0)),
            pl.BlockSpec((IB, D), lambda i: (i, 0)),
        ],
        out_shape=[
            jax.ShapeDtypeStruct((N, 1), jnp.int32),
            jax.ShapeDtypeStruct((N, D), jnp.float32),
        ],
    )(v_row, v_col, h)

    rank_row = rank_i.reshape(1, N)

    a = pl.pallas_call(
        _build_a_kernel,
        grid=(N // AB,),
        in_specs=[
            pl.BlockSpec((1, N), lambda i: (0, 0)),
            pl.BlockSpec((N, AB), lambda i: (0, i)),
        ],
        out_specs=pl.BlockSpec((K, AB), lambda i: (0, i)),
        out_shape=jax.ShapeDtypeStruct((K, N), jnp.bfloat16),
        scratch_shapes=[pltpu.VMEM((K, N), jnp.bfloat16)],
    )(rank_row, g)

    g_out, idx_row = pl.pallas_call(
        _twohop_kernel,
        grid=(N // MB,),
        in_specs=[
            pl.BlockSpec((K, N), lambda m: (0, 0)),
            pl.BlockSpec((N, MB), lambda m: (0, m)),
            pl.BlockSpec((MB, 1), lambda m: (m, 0)),
        ],
        out_specs=[
            pl.BlockSpec((K, K), lambda m: (0, 0)),
            pl.BlockSpec((1, K), lambda m: (0, 0)),
        ],
        out_shape=[
            jax.ShapeDtypeStruct((K, K), jnp.float32),
            jax.ShapeDtypeStruct((1, K), jnp.int32),
        ],
        scratch_shapes=[pltpu.VMEM((1, K), jnp.float32)],
    )(a, g, rank_i)

    new_h_full = _sc_scatter_rows(hv, rank_i)

    return (g_out, new_h_full[:K], idx_row.reshape(K))


# speedup vs baseline: 1.7635x; 1.7635x over previous
import jax
import jax.numpy as jnp
from jax.experimental import pallas as pl
from jax.experimental.pallas import tpu as pltpu
from jax.experimental.pallas import tpu_sc as plsc

N = 4096
D = 128
K = 2048

IB = 512
AB = 512
MB = 512
WIN = 128

F8 = jnp.float8_e4m3fn


def _rank_kernel(v_row_ref, v_col_ref, h_ref, rank_i_ref, hv_ref):
    pid = pl.program_id(0)
    v_row = v_row_ref[...]
    v_col = v_col_ref[...]
    gt = v_row > v_col
    eq = v_row == v_col
    jj = jax.lax.broadcasted_iota(jnp.int32, (IB, N), 1)
    ii = jax.lax.broadcasted_iota(jnp.int32, (IB, N), 0) + pid * IB
    cnt = gt.astype(jnp.float32) + (eq & (jj < ii)).astype(jnp.float32)
    rank = jnp.sum(cnt, axis=1, keepdims=True)
    rank_i_ref[...] = rank.astype(jnp.int32)
    hv_ref[...] = h_ref[...] * v_col


def _build_a_kernel(rank_row_ref, g_ref, a_ref, m_scratch):
    pid = pl.program_id(0)

    @pl.when(pid == 0)
    def _():
        rank_row = rank_row_ref[...]
        for rc in range(K // IB):
            rr = jax.lax.broadcasted_iota(jnp.int32, (IB, N), 0) + (rc * IB)
            m_scratch[rc * IB:(rc + 1) * IB, :] = (rank_row == rr).astype(F8)

    gb = (g_ref[...] != 0).astype(F8)
    a_ref[...] = jnp.dot(m_scratch[...], gb,
                         preferred_element_type=jnp.float32).astype(F8)


def _twohop_kernel(a_ref, g_ref, rank_col_ref, out_ref, idx_ref, idxacc):
    m = pl.program_id(0)
    nm = pl.num_programs(0)

    @pl.when(m == 0)
    def _():
        out_ref[...] = jnp.zeros_like(out_ref)
        idxacc[...] = jnp.zeros_like(idxacc)

    gb = (g_ref[...] != 0).astype(F8)
    d = jnp.dot(a_ref[...], gb, preferred_element_type=jnp.float32)
    dbool = (d != 0.0).astype(F8)

    rank_col = rank_col_ref[...]
    jcols = jax.lax.broadcasted_iota(jnp.int32, (MB, K), 1)
    p_f = (rank_col == jcols).astype(jnp.float32)
    out_ref[...] += jnp.dot(dbool, p_f.astype(F8),
                            preferred_element_type=jnp.float32)

    miota = (jax.lax.broadcasted_iota(jnp.int32, (MB, 1), 0)
             + m * MB).astype(jnp.float32)
    idxacc[...] += jnp.sum(p_f * miota, axis=0, keepdims=True)
    idx_ref[...] = idxacc[...].astype(jnp.int32)

    @pl.when(m == nm - 1)
    def _():
        c = out_ref[...]
        ones = jnp.ones((1, K), jnp.float32)
        deg = jax.lax.dot_general(ones, c, (((1,), (1,)), ((), ())),
                                  preferred_element_type=jnp.float32)
        out_ref[...] = c / deg


def _sc_scatter_rows(hv, rank_i32):
    rank2 = rank_i32.reshape(1, N)
    mesh = plsc.VectorSubcoreMesh(core_axis_name="c", subcore_axis_name="s")

    @pl.kernel(out_type=jax.ShapeDtypeStruct((N, D), jnp.float32), mesh=mesh)
    def k(hv_hbm, r_hbm, o_hbm):
        def body(hv_vmem, r_vmem):
            pltpu.sync_copy(hv_vmem, o_hbm.at[r_vmem.at[0]])

        pltpu.emit_pipeline(
            body,
            grid=(N // WIN,),
            in_specs=[pl.BlockSpec((WIN, D), lambda i: (i, 0)),
                      pl.BlockSpec((1, WIN), lambda i: (0, i))],
            out_specs=[],
            core_axis_name=("c", "s"),
            dimension_semantics=(pltpu.PARALLEL,),
        )(hv_hbm, r_hbm)

    return k(hv, rank2)


def kernel(g, h, W, b):
    weights = (h @ W + b).squeeze(-1)
    v = jax.nn.sigmoid(weights)
    v_row = v.reshape(1, N)
    v_col = v.reshape(N, 1)

    rank_i, hv = pl.pallas_call(
        _rank_kernel,
        grid=(N // IB,),
        in_specs=[
            pl.BlockSpec((1, N), lambda i: (0, 0)),
            pl.BlockSpec((IB, 1), lambda i: (i, 0)),
            pl.BlockSpec((IB, D), lambda i: (i, 0)),
        ],
        out_specs=[
            pl.BlockSpec((IB, 1), lambda i: (i, 0)),
            pl.BlockSpec((IB, D), lambda i: (i, 0)),
        ],
        out_shape=[
            jax.ShapeDtypeStruct((N, 1), jnp.int32),
            jax.ShapeDtypeStruct((N, D), jnp.float32),
        ],
    )(v_row, v_col, h)

    rank_row = rank_i.reshape(1, N)

    a = pl.pallas_call(
        _build_a_kernel,
        grid=(N // AB,),
        in_specs=[
            pl.BlockSpec((1, N), lambda i: (0, 0)),
            pl.BlockSpec((N, AB), lambda i: (0, i)),
        ],
        out_specs=pl.BlockSpec((K, AB), lambda i: (0, i)),
        out_shape=jax.ShapeDtypeStruct((K, N), F8),
        scratch_shapes=[pltpu.VMEM((K, N), F8)],
    )(rank_row, g)

    g_out, idx_row = pl.pallas_call(
        _twohop_kernel,
        grid=(N // MB,),
        in_specs=[
            pl.BlockSpec((K, N), lambda m: (0, 0)),
            pl.BlockSpec((N, MB), lambda m: (0, m)),
            pl.BlockSpec((MB, 1), lambda m: (m, 0)),
        ],
        out_specs=[
            pl.BlockSpec((K, K), lambda m: (0, 0)),
            pl.BlockSpec((1, K), lambda m: (0, 0)),
        ],
        out_shape=[
            jax.ShapeDtypeStruct((K, K), jnp.float32),
            jax.ShapeDtypeStruct((1, K), jnp.int32),
        ],
        scratch_shapes=[pltpu.VMEM((1, K), jnp.float32)],
    )(a, g, rank_i)

    new_h_full = _sc_scatter_rows(hv, rank_i)

    return (g_out, new_h_full[:K], idx_row.reshape(K))
